# baseline (speedup 1.0000x reference)
import jax
import jax.numpy as jnp
from jax.experimental import pallas as pl

_BLK = 1024
_H = 64


def _pad_rows(a, npad):
    return jnp.pad(a, ((0, npad - a.shape[0]),) + ((0, 0),) * (a.ndim - 1))


def _k_stage1(x_ref, gb_ref, win_ref, bin_ref, w1_ref, o_ref):
    h = jnp.dot(x_ref[...], win_ref[...], preferred_element_type=jnp.float32)
    h = h + bin_ref[...]
    gb = gb_ref[...]
    h = h * (1.0 + gb[:, :_H]) + gb[:, _H:]
    o_ref[...] = jnp.dot(h, w1_ref[...], preferred_element_type=jnp.float32)


def _k_stage2(a_ref, gb_ref, b1_ref, w2_ref, o_ref):
    h = jnp.maximum(a_ref[...] + b1_ref[...], 0.0)
    gb = gb_ref[...]
    h = h * (1.0 + gb[:, :_H]) + gb[:, _H:]
    o_ref[...] = jnp.dot(h, w2_ref[...], preferred_element_type=jnp.float32)


def _k_stage3(a_ref, x_ref, b2_ref, cw1_ref, cb1_ref, cw2_ref, cb2_ref,
              fw1_ref, fb1_ref, fw2_ref, o_ref):
    h = jnp.maximum(a_ref[...] + b2_ref[...], 0.0)
    z = jnp.dot(h, cw1_ref[...], preferred_element_type=jnp.float32) + cb1_ref[...]
    z = jnp.maximum(z, 0.0)
    logits = jnp.dot(z, cw2_ref[...], preferred_element_type=jnp.float32) + cb2_ref[...]
    f = jnp.dot(x_ref[...], fw1_ref[...], preferred_element_type=jnp.float32) + fb1_ref[...]
    f = jnp.maximum(f, 0.0)
    fl = jnp.dot(f, fw2_ref[...], preferred_element_type=jnp.float32)
    o_ref[...] = logits + 0.03 * fl


def _edge_agg(xw, src, dst, n, a_s, a_d):
    asrc = xw @ a_s
    adst = xw @ a_d
    e = asrc[src] + adst[dst]
    e = jnp.where(e >= 0, e, 0.2 * e)
    emax = jax.ops.segment_max(e, dst, num_segments=n)
    emax = jnp.where(jnp.isfinite(emax), emax, 0.0)
    ex = jnp.exp(e - emax[dst])
    den = jax.ops.segment_sum(ex, dst, num_segments=n)
    alpha = ex / (den[dst] + 1e-16)
    return jax.ops.segment_sum(xw[src] * alpha[:, None], dst, num_segments=n)


def kernel(x, edge_index, batch, climber, params):
    p = params
    n = x.shape[0]
    npad = ((n + _BLK - 1) // _BLK) * _BLK
    grid = npad // _BLK

    mu = climber.mean(-1, keepdims=True)
    var = climber.var(-1, keepdims=True)
    cn = (climber - mu) / jnp.sqrt(var + 1e-5) * p['ln_g'] + p['ln_b']
    c = jax.nn.relu(cn @ p['ce_W'] + p['ce_b'])
    gb1 = (c @ p['film1_W'] + p['film1_b'])[batch]
    gb2 = (c @ p['film2_W'] + p['film2_b'])[batch]

    xp = _pad_rows(x, npad)
    gb1p = _pad_rows(gb1, npad)
    gb2p = _pad_rows(gb2, npad)

    win = jnp.zeros((8, _H), jnp.float32).at[:6].set(p['lin_in_W'])
    bin_ = p['lin_in_b'][None]

    full = lambda s: pl.BlockSpec(s, lambda i: (0,) * len(s))
    row = lambda w: pl.BlockSpec((_BLK, w), lambda i: (i, 0))

    x1 = pl.pallas_call(
        _k_stage1,
        grid=(grid,),
        in_specs=[row(8), row(2 * _H), full((8, _H)), full((1, _H)),
                  full((_H, _H))],
        out_specs=row(_H),
        out_shape=jax.ShapeDtypeStruct((npad, _H), jnp.float32),
    )(xp, gb1p, win, bin_, p['W1'])[:n]

    loop = jnp.arange(n, dtype=edge_index.dtype)
    src = jnp.concatenate([edge_index[0], loop])
    dst = jnp.concatenate([edge_index[1], loop])

    agg1 = _edge_agg(x1, src, dst, n, p['as1'], p['ad1'])

    x2 = pl.pallas_call(
        _k_stage2,
        grid=(grid,),
        in_specs=[row(_H), row(2 * _H), full((1, _H)), full((_H, _H))],
        out_specs=row(_H),
        out_shape=jax.ShapeDtypeStruct((npad, _H), jnp.float32),
    )(_pad_rows(agg1, npad), gb2p, p['b1'][None], p['W2'])[:n]

    agg2 = _edge_agg(x2, src, dst, n, p['as2'], p['ad2'])

    cw2 = jnp.zeros((_H, 128), jnp.float32).at[:, :4].set(p['cls_W2'])
    cb2 = jnp.zeros((1, 128), jnp.float32).at[0, :4].set(
        p['cls_b2'] + 0.03 * p['fg_b2'])
    fw1 = jnp.zeros((8, 8), jnp.float32).at[6:8].set(p['fg_W1'])
    fb1 = p['fg_b1'][None]
    fw2 = jnp.zeros((8, 128), jnp.float32).at[:, :4].set(p['fg_W2'])

    out = pl.pallas_call(
        _k_stage3,
        grid=(grid,),
        in_specs=[row(_H), row(8), full((1, _H)), full((_H, _H)),
                  full((1, _H)), full((_H, 128)), full((1, 128)),
                  full((8, 8)), full((1, 8)), full((8, 128))],
        out_specs=row(128),
        out_shape=jax.ShapeDtypeStruct((npad, 128), jnp.float32),
    )(_pad_rows(agg2, npad), xp, p['b2'][None], p['cls_W1'],
      p['cls_b1'][None], cw2, cb2, fw1, fb1, fw2)

    return out[:n, :4]
